# baseline (speedup 1.0000x reference)
import functools

import jax
import jax.numpy as jnp
from jax import lax
from jax.experimental import pallas as pl
from jax.experimental.pallas import tpu as pltpu
from jax.experimental.pallas import tpu_sc as plsc

NB_LAYER = 2
K = 8
N = 4096
HD = 256
ROW_BLK = 256
N_BLK = N // ROW_BLK

SC_NC = 2
SC_NS = 16
SC_NW = SC_NC * SC_NS
NPW = N // SC_NW
CH = 16
NSTEP = NPW // CH


def _prep_body(x_ref, xsp_ref, w1_ref, b1_ref, ws_ref, bs_ref, h_ref, xs_ref):
    h_ref[...] = (
        jnp.dot(x_ref[...], w1_ref[...], preferred_element_type=jnp.float32)
        + b1_ref[...]
    )
    xs_ref[...] = (
        jnp.dot(xsp_ref[...], ws_ref[...], preferred_element_type=jnp.float32)
        + bs_ref[...]
    )


def _topk_body(xs_ref, xst_ref, noise_ref, scale_ref, lp_ref, idx_ref):
    scale = scale_ref[0, 0, 0]
    xs_blk = xs_ref[...]
    xst = xst_ref[...]
    sq_all = jnp.sum(xst * xst, axis=0)[None, :]
    sq_blk = jnp.sum(xs_blk * xs_blk, axis=1)[:, None]
    prod = jnp.dot(xs_blk, xst, preferred_element_type=jnp.float32)
    ds = jnp.maximum(sq_blk + sq_all - 2.0 * prod, 0.0) * scale
    q = noise_ref[0]
    v = jnp.log(-jnp.log(q)) - ds
    nn = v.shape[1]
    ii = lax.broadcasted_iota(jnp.int32, (nn, 2), 0)
    cc = lax.broadcasted_iota(jnp.int32, (nn, 2), 1)
    digits = jnp.where(cc == 0, ii >> 6, ii & 63).astype(jnp.float32)
    neg = jnp.float32(-jnp.inf)
    vals, idxs = [], []
    for k in range(K):
        m = jnp.max(v, axis=1, keepdims=True)
        match = v >= m
        matchf = match.astype(jnp.float32)
        am2 = jnp.dot(matchf, digits, preferred_element_type=jnp.float32)
        vals.append(m)
        idxs.append(am2[:, 0:1] * 64.0 + am2[:, 1:2])
        if k < K - 1:
            v = jnp.where(match, neg, v)
    lp_ref[...] = jnp.concatenate(vals, axis=1)
    idx_ref[...] = jnp.clip(
        jnp.concatenate(idxs, axis=1).astype(jnp.int32), 0, nn - 1
    )


def _sc_gather_body(h_hbm, idx_hbm, out_hbm, idx_v, rows0, rows1, acc_v,
                    sem0, sem1):
    wid = lax.axis_index("s") * SC_NC + lax.axis_index("c")
    base = wid * NPW
    pltpu.sync_copy(idx_hbm.at[wid], idx_v)
    idx2 = idx_v

    def reduce_chunk(rows_v, st):
        def per_dst(i, c2):
            r = i * K
            for f in range(HD // 16):
                sl = pl.ds(f * 16, 16)
                s0 = rows_v[r + 0, sl] + rows_v[r + 1, sl]
                s1 = rows_v[r + 2, sl] + rows_v[r + 3, sl]
                s2 = rows_v[r + 4, sl] + rows_v[r + 5, sl]
                s3 = rows_v[r + 6, sl] + rows_v[r + 7, sl]
                acc_v[st * CH + i, sl] = (s0 + s1) + (s2 + s3)
            return c2

        lax.fori_loop(0, CH, per_dst, 0)

    pltpu.async_copy(h_hbm.at[idx2.at[0]], rows0, sem0)

    def pair(p, carry):
        s0 = 2 * p
        s1 = s0 + 1
        pltpu.async_copy(h_hbm.at[idx2.at[s1]], rows1, sem1)
        pltpu.make_async_copy(h_hbm.at[idx2.at[s0]], rows0, sem0).wait()
        reduce_chunk(rows0, s0)

        @pl.when(s0 + 2 < NSTEP)
        def _():
            pltpu.async_copy(h_hbm.at[idx2.at[s0 + 2]], rows0, sem0)

        pltpu.make_async_copy(h_hbm.at[idx2.at[s1]], rows1, sem1).wait()
        reduce_chunk(rows1, s1)
        return carry

    lax.fori_loop(0, NSTEP // 2, pair, 0)
    pltpu.sync_copy(acc_v, out_hbm.at[pl.ds(base, NPW)])


def _gather_mean(h, idx_flat):
    mesh = plsc.VectorSubcoreMesh(core_axis_name="c", subcore_axis_name="s")
    fn = functools.partial(
        pl.kernel,
        mesh=mesh,
        out_type=jax.ShapeDtypeStruct((N, HD), jnp.float32),
        scratch_types=[
            pltpu.VMEM((NSTEP, CH * K), jnp.int32),
            pltpu.VMEM((CH * K, HD), jnp.float32),
            pltpu.VMEM((CH * K, HD), jnp.float32),
            pltpu.VMEM((NPW, HD), jnp.float32),
            pltpu.SemaphoreType.DMA,
            pltpu.SemaphoreType.DMA,
        ],
    )(_sc_gather_body)
    return fn(h, idx_flat.reshape(SC_NW, NSTEP, CH * K))


def _mm_body(g_ref, wg_ref, bg_ref, out_ref):
    g = g_ref[...] * jnp.float32(1.0 / K)
    out_ref[...] = (
        jnp.dot(g, wg_ref[...], preferred_element_type=jnp.float32)
        + bg_ref[...]
    )


def _mm_final_body(g_ref, wg_ref, bg_ref, wl_ref, bl_ref, out_ref):
    g = g_ref[...] * jnp.float32(1.0 / K)
    h2 = (
        jnp.dot(g, wg_ref[...], preferred_element_type=jnp.float32)
        + bg_ref[...]
    )
    out_ref[...] = (
        jnp.dot(h2, wl_ref[...], preferred_element_type=jnp.float32)
        + bl_ref[...]
    )


def kernel(x, x_spatial, W1, b1, Ws, bs, Wl, bl, Wg, bg, temp, noise):
    n = x.shape[0]
    hdim = W1.shape[1]
    odim = Wl.shape[1]

    h, xs = pl.pallas_call(
        _prep_body,
        out_shape=(
            jax.ShapeDtypeStruct((n, hdim), jnp.float32),
            jax.ShapeDtypeStruct((n, hdim), jnp.float32),
        ),
    )(x, x_spatial, W1, b1.reshape(1, hdim), Ws, bs.reshape(1, hdim))

    xst = xs.T
    scale = jnp.exp(jnp.clip(temp, -5.0, 5.0)).reshape(NB_LAYER, 1, 1)

    def topk_layer(i):
        return pl.pallas_call(
            _topk_body,
            grid=(N_BLK,),
            in_specs=[
                pl.BlockSpec((ROW_BLK, hdim), lambda b: (b, 0)),
                pl.BlockSpec((hdim, n), lambda b: (0, 0)),
                pl.BlockSpec((1, ROW_BLK, n), lambda b, _l=i: (_l, b, 0)),
                pl.BlockSpec((1, 1, 1), lambda b, _l=i: (_l, 0, 0)),
            ],
            out_specs=(
                pl.BlockSpec((ROW_BLK, K), lambda b: (b, 0)),
                pl.BlockSpec((ROW_BLK, K), lambda b: (b, 0)),
            ),
            out_shape=(
                jax.ShapeDtypeStruct((n, K), jnp.float32),
                jax.ShapeDtypeStruct((n, K), jnp.int32),
            ),
        )(xs, xst, noise, scale)

    lp0, idx0 = topk_layer(0)
    g0 = _gather_mean(h, idx0.reshape(-1))
    lp1, idx1 = topk_layer(1)
    h1 = pl.pallas_call(
        _mm_body,
        out_shape=jax.ShapeDtypeStruct((n, hdim), jnp.float32),
    )(g0, Wg[0], bg[0].reshape(1, hdim))

    g1 = _gather_mean(h1, idx1.reshape(-1))
    out = pl.pallas_call(
        _mm_final_body,
        out_shape=jax.ShapeDtypeStruct((n, odim), jnp.float32),
    )(g1, Wg[1], bg[1].reshape(1, hdim), Wl, bl.reshape(1, odim))

    rows = jnp.tile(jnp.arange(n, dtype=jnp.int32)[:, None], (1, K)).reshape(-1)
    edges = tuple(
        jnp.stack([i.reshape(-1), rows], axis=0) for i in (idx0, idx1)
    )
    return (out, (lp0, lp1), edges)


# speedup vs baseline: 1.0906x; 1.0906x over previous
import functools

import jax
import jax.numpy as jnp
from jax import lax
from jax.experimental import pallas as pl
from jax.experimental.pallas import tpu as pltpu
from jax.experimental.pallas import tpu_sc as plsc

NB_LAYER = 2
K = 8
N = 4096
HD = 256
ROW_BLK = 256
N_BLK = N // ROW_BLK

SC_NC = 2
SC_NS = 16
SC_NW = SC_NC * SC_NS
NPW = N // SC_NW
CH = 16
NSTEP = NPW // CH


def _prep_body(x_ref, xsp_ref, w1_ref, b1_ref, ws_ref, bs_ref, h_ref, xs_ref):
    h_ref[...] = (
        jnp.dot(x_ref[...], w1_ref[...], preferred_element_type=jnp.float32)
        + b1_ref[...]
    )
    xs_ref[...] = (
        jnp.dot(xsp_ref[...], ws_ref[...], preferred_element_type=jnp.float32)
        + bs_ref[...]
    )


def _topk_body(xs_ref, xst_ref, noise_ref, scale_ref, lp_ref, idx_ref):
    scale = scale_ref[0, 0, 0]
    xs_blk = xs_ref[...]
    xst = xst_ref[...]
    sq_all = jnp.sum(xst * xst, axis=0)[None, :]
    sq_blk = jnp.sum(xs_blk * xs_blk, axis=1)[:, None]
    prod = jnp.dot(xs_blk, xst, preferred_element_type=jnp.float32)
    ds = jnp.maximum(sq_blk + sq_all - 2.0 * prod, 0.0) * scale
    q = noise_ref[0]
    v = jnp.log(-jnp.log(q)) - ds
    iota_f = lax.broadcasted_iota(jnp.int32, v.shape, 1).astype(jnp.float32)
    neg = jnp.float32(-jnp.inf)
    big = jnp.float32(2.0 * N)
    vals, idxs = [], []
    for k in range(K):
        m = jnp.max(v, axis=1, keepdims=True)
        match = v >= m
        am = jnp.min(jnp.where(match, iota_f, big), axis=1, keepdims=True)
        vals.append(m)
        idxs.append(am)
        if k < K - 1:
            v = jnp.where(match, neg, v)
    lp_ref[...] = jnp.concatenate(vals, axis=1)
    idx_ref[...] = jnp.concatenate(idxs, axis=1).astype(jnp.int32)


def _sc_gather_body(h_hbm, idx_hbm, out_hbm, idx_v, rows0, rows1, acc_v,
                    sem0, sem1):
    wid = lax.axis_index("s") * SC_NC + lax.axis_index("c")
    base = wid * NPW
    pltpu.sync_copy(idx_hbm.at[wid], idx_v)
    idx2 = idx_v

    def reduce_chunk(rows_v, st):
        def per_dst(i, c2):
            r = i * K
            for f in range(HD // 16):
                sl = pl.ds(f * 16, 16)
                s0 = rows_v[r + 0, sl] + rows_v[r + 1, sl]
                s1 = rows_v[r + 2, sl] + rows_v[r + 3, sl]
                s2 = rows_v[r + 4, sl] + rows_v[r + 5, sl]
                s3 = rows_v[r + 6, sl] + rows_v[r + 7, sl]
                acc_v[st * CH + i, sl] = (s0 + s1) + (s2 + s3)
            return c2

        lax.fori_loop(0, CH, per_dst, 0)

    pltpu.async_copy(h_hbm.at[idx2.at[0]], rows0, sem0)

    def pair(p, carry):
        s0 = 2 * p
        s1 = s0 + 1
        pltpu.async_copy(h_hbm.at[idx2.at[s1]], rows1, sem1)
        pltpu.make_async_copy(h_hbm.at[idx2.at[s0]], rows0, sem0).wait()
        reduce_chunk(rows0, s0)

        @pl.when(s0 + 2 < NSTEP)
        def _():
            pltpu.async_copy(h_hbm.at[idx2.at[s0 + 2]], rows0, sem0)

        pltpu.make_async_copy(h_hbm.at[idx2.at[s1]], rows1, sem1).wait()
        reduce_chunk(rows1, s1)
        return carry

    lax.fori_loop(0, NSTEP // 2, pair, 0)
    pltpu.sync_copy(acc_v, out_hbm.at[pl.ds(base, NPW)])


def _gather_mean(h, idx_flat):
    mesh = plsc.VectorSubcoreMesh(core_axis_name="c", subcore_axis_name="s")
    fn = functools.partial(
        pl.kernel,
        mesh=mesh,
        out_type=jax.ShapeDtypeStruct((N, HD), jnp.float32),
        scratch_types=[
            pltpu.VMEM((NSTEP, CH * K), jnp.int32),
            pltpu.VMEM((CH * K, HD), jnp.float32),
            pltpu.VMEM((CH * K, HD), jnp.float32),
            pltpu.VMEM((NPW, HD), jnp.float32),
            pltpu.SemaphoreType.DMA,
            pltpu.SemaphoreType.DMA,
        ],
    )(_sc_gather_body)
    return fn(h, idx_flat.reshape(SC_NW, NSTEP, CH * K))


def _mm_body(g_ref, wg_ref, bg_ref, out_ref):
    g = g_ref[...] * jnp.float32(1.0 / K)
    out_ref[...] = (
        jnp.dot(g, wg_ref[...], preferred_element_type=jnp.float32)
        + bg_ref[...]
    )


def _mm_final_body(g_ref, wg_ref, bg_ref, wl_ref, bl_ref, out_ref):
    g = g_ref[...] * jnp.float32(1.0 / K)
    h2 = (
        jnp.dot(g, wg_ref[...], preferred_element_type=jnp.float32)
        + bg_ref[...]
    )
    out_ref[...] = (
        jnp.dot(h2, wl_ref[...], preferred_element_type=jnp.float32)
        + bl_ref[...]
    )


def kernel(x, x_spatial, W1, b1, Ws, bs, Wl, bl, Wg, bg, temp, noise):
    n = x.shape[0]
    hdim = W1.shape[1]
    odim = Wl.shape[1]

    h, xs = pl.pallas_call(
        _prep_body,
        out_shape=(
            jax.ShapeDtypeStruct((n, hdim), jnp.float32),
            jax.ShapeDtypeStruct((n, hdim), jnp.float32),
        ),
    )(x, x_spatial, W1, b1.reshape(1, hdim), Ws, bs.reshape(1, hdim))

    xst = xs.T
    scale = jnp.exp(jnp.clip(temp, -5.0, 5.0)).reshape(NB_LAYER, 1, 1)

    def topk_layer(i):
        return pl.pallas_call(
            _topk_body,
            grid=(N_BLK,),
            in_specs=[
                pl.BlockSpec((ROW_BLK, hdim), lambda b: (b, 0)),
                pl.BlockSpec((hdim, n), lambda b: (0, 0)),
                pl.BlockSpec((1, ROW_BLK, n), lambda b, _l=i: (_l, b, 0)),
                pl.BlockSpec((1, 1, 1), lambda b, _l=i: (_l, 0, 0)),
            ],
            out_specs=(
                pl.BlockSpec((ROW_BLK, K), lambda b: (b, 0)),
                pl.BlockSpec((ROW_BLK, K), lambda b: (b, 0)),
            ),
            out_shape=(
                jax.ShapeDtypeStruct((n, K), jnp.float32),
                jax.ShapeDtypeStruct((n, K), jnp.int32),
            ),
        )(xs, xst, noise, scale)

    lp0, idx0 = topk_layer(0)
    g0 = _gather_mean(h, idx0.reshape(-1))
    lp1, idx1 = topk_layer(1)
    h1 = pl.pallas_call(
        _mm_body,
        out_shape=jax.ShapeDtypeStruct((n, hdim), jnp.float32),
    )(g0, Wg[0], bg[0].reshape(1, hdim))

    g1 = _gather_mean(h1, idx1.reshape(-1))
    out = pl.pallas_call(
        _mm_final_body,
        out_shape=jax.ShapeDtypeStruct((n, odim), jnp.float32),
    )(g1, Wg[1], bg[1].reshape(1, hdim), Wl, bl.reshape(1, odim))

    rows = jnp.tile(jnp.arange(n, dtype=jnp.int32)[:, None], (1, K)).reshape(-1)
    edges = tuple(
        jnp.stack([i.reshape(-1), rows], axis=0) for i in (idx0, idx1)
    )
    return (out, (lp0, lp1), edges)


# speedup vs baseline: 1.1309x; 1.0370x over previous
import functools

import jax
import jax.numpy as jnp
from jax import lax
from jax.experimental import pallas as pl
from jax.experimental.pallas import tpu as pltpu
from jax.experimental.pallas import tpu_sc as plsc

NB_LAYER = 2
K = 8
N = 4096
HD = 256
ROW_BLK = 256
N_BLK = N // ROW_BLK

SC_NC = 2
SC_NS = 16
SC_NW = SC_NC * SC_NS
NPW = N // SC_NW
CH = 16
NSTEP = NPW // CH


def _prep_body(x_ref, xsp_ref, w1_ref, b1_ref, ws_ref, bs_ref, h_ref, xs_ref):
    h_ref[...] = (
        jnp.dot(x_ref[...], w1_ref[...], preferred_element_type=jnp.float32)
        + b1_ref[...]
    )
    xs_ref[...] = (
        jnp.dot(xsp_ref[...], ws_ref[...], preferred_element_type=jnp.float32)
        + bs_ref[...]
    )


def _topk_body(xs_ref, xst_ref, n0_ref, n1_ref, scale_ref,
               lp0_ref, idx0_ref, lp1_ref, idx1_ref):
    xs_blk = xs_ref[...]
    xst = xst_ref[...]
    sq_all = jnp.sum(xst * xst, axis=0)[None, :]
    sq_blk = jnp.sum(xs_blk * xs_blk, axis=1)[:, None]
    prod = jnp.dot(xs_blk, xst, preferred_element_type=jnp.float32)
    d = jnp.maximum(sq_blk + sq_all - 2.0 * prod, 0.0)
    iota_f = lax.broadcasted_iota(jnp.int32, d.shape, 1).astype(jnp.float32)
    neg = jnp.float32(-jnp.inf)
    big = jnp.float32(2.0 * N)

    for l, (n_ref, lp_ref, idx_ref) in enumerate(
        [(n0_ref, lp0_ref, idx0_ref), (n1_ref, lp1_ref, idx1_ref)]
    ):
        v = jnp.log(-jnp.log(n_ref[0])) - d * scale_ref[l, 0, 0]
        vals, idxs = [], []
        for k in range(K):
            m = jnp.max(v, axis=1, keepdims=True)
            match = v >= m
            am = jnp.min(jnp.where(match, iota_f, big), axis=1, keepdims=True)
            vals.append(m)
            idxs.append(am)
            if k < K - 1:
                v = jnp.where(match, neg, v)
        lp_ref[...] = jnp.concatenate(vals, axis=1)
        idx_ref[...] = jnp.concatenate(idxs, axis=1).astype(jnp.int32)


def _sc_gather_body(h_hbm, idx_hbm, out_hbm, idx_v, rows0, rows1, acc_v,
                    sem0, sem1):
    wid = lax.axis_index("s") * SC_NC + lax.axis_index("c")
    base = wid * NPW
    pltpu.sync_copy(idx_hbm.at[wid], idx_v)
    idx2 = idx_v

    def reduce_chunk(rows_v, st):
        def per_dst(i, c2):
            r = i * K
            for f in range(HD // 16):
                sl = pl.ds(f * 16, 16)
                s0 = rows_v[r + 0, sl] + rows_v[r + 1, sl]
                s1 = rows_v[r + 2, sl] + rows_v[r + 3, sl]
                s2 = rows_v[r + 4, sl] + rows_v[r + 5, sl]
                s3 = rows_v[r + 6, sl] + rows_v[r + 7, sl]
                acc_v[st * CH + i, sl] = (s0 + s1) + (s2 + s3)
            return c2

        lax.fori_loop(0, CH, per_dst, 0)

    pltpu.async_copy(h_hbm.at[idx2.at[0]], rows0, sem0)

    def pair(p, carry):
        s0 = 2 * p
        s1 = s0 + 1
        pltpu.async_copy(h_hbm.at[idx2.at[s1]], rows1, sem1)
        pltpu.make_async_copy(h_hbm.at[idx2.at[s0]], rows0, sem0).wait()
        reduce_chunk(rows0, s0)

        @pl.when(s0 + 2 < NSTEP)
        def _():
            pltpu.async_copy(h_hbm.at[idx2.at[s0 + 2]], rows0, sem0)

        pltpu.make_async_copy(h_hbm.at[idx2.at[s1]], rows1, sem1).wait()
        reduce_chunk(rows1, s1)
        return carry

    lax.fori_loop(0, NSTEP // 2, pair, 0)
    pltpu.sync_copy(acc_v, out_hbm.at[pl.ds(base, NPW)])


def _gather_mean(h, idx_flat):
    mesh = plsc.VectorSubcoreMesh(core_axis_name="c", subcore_axis_name="s")
    fn = functools.partial(
        pl.kernel,
        mesh=mesh,
        out_type=jax.ShapeDtypeStruct((N, HD), jnp.float32),
        scratch_types=[
            pltpu.VMEM((NSTEP, CH * K), jnp.int32),
            pltpu.VMEM((CH * K, HD), jnp.float32),
            pltpu.VMEM((CH * K, HD), jnp.float32),
            pltpu.VMEM((NPW, HD), jnp.float32),
            pltpu.SemaphoreType.DMA,
            pltpu.SemaphoreType.DMA,
        ],
    )(_sc_gather_body)
    return fn(h, idx_flat.reshape(SC_NW, NSTEP, CH * K))


def _mm_body(g_ref, wg_ref, bg_ref, out_ref):
    g = g_ref[...] * jnp.float32(1.0 / K)
    out_ref[...] = (
        jnp.dot(g, wg_ref[...], preferred_element_type=jnp.float32)
        + bg_ref[...]
    )


def _mm_final_body(g_ref, wg_ref, bg_ref, wl_ref, bl_ref, out_ref):
    g = g_ref[...] * jnp.float32(1.0 / K)
    h2 = (
        jnp.dot(g, wg_ref[...], preferred_element_type=jnp.float32)
        + bg_ref[...]
    )
    out_ref[...] = (
        jnp.dot(h2, wl_ref[...], preferred_element_type=jnp.float32)
        + bl_ref[...]
    )


def kernel(x, x_spatial, W1, b1, Ws, bs, Wl, bl, Wg, bg, temp, noise):
    n = x.shape[0]
    hdim = W1.shape[1]
    odim = Wl.shape[1]

    h, xs = pl.pallas_call(
        _prep_body,
        out_shape=(
            jax.ShapeDtypeStruct((n, hdim), jnp.float32),
            jax.ShapeDtypeStruct((n, hdim), jnp.float32),
        ),
    )(x, x_spatial, W1, b1.reshape(1, hdim), Ws, bs.reshape(1, hdim))

    xst = xs.T
    scale = jnp.exp(jnp.clip(temp, -5.0, 5.0)).reshape(NB_LAYER, 1, 1)

    okspec = pl.BlockSpec((ROW_BLK, K), lambda b: (b, 0))
    lp0, idx0, lp1, idx1 = pl.pallas_call(
        _topk_body,
        grid=(N_BLK,),
        in_specs=[
            pl.BlockSpec((ROW_BLK, hdim), lambda b: (b, 0)),
            pl.BlockSpec((hdim, n), lambda b: (0, 0)),
            pl.BlockSpec((1, ROW_BLK, n), lambda b: (0, b, 0)),
            pl.BlockSpec((1, ROW_BLK, n), lambda b: (1, b, 0)),
            pl.BlockSpec((NB_LAYER, 1, 1), lambda b: (0, 0, 0)),
        ],
        out_specs=(okspec, okspec, okspec, okspec),
        out_shape=(
            jax.ShapeDtypeStruct((n, K), jnp.float32),
            jax.ShapeDtypeStruct((n, K), jnp.int32),
            jax.ShapeDtypeStruct((n, K), jnp.float32),
            jax.ShapeDtypeStruct((n, K), jnp.int32),
        ),
    )(xs, xst, noise, noise, scale)

    g0 = _gather_mean(h, idx0.reshape(-1))
    h1 = pl.pallas_call(
        _mm_body,
        out_shape=jax.ShapeDtypeStruct((n, hdim), jnp.float32),
    )(g0, Wg[0], bg[0].reshape(1, hdim))

    g1 = _gather_mean(h1, idx1.reshape(-1))
    out = pl.pallas_call(
        _mm_final_body,
        out_shape=jax.ShapeDtypeStruct((n, odim), jnp.float32),
    )(g1, Wg[1], bg[1].reshape(1, hdim), Wl, bl.reshape(1, odim))

    rows = jnp.tile(jnp.arange(n, dtype=jnp.int32)[:, None], (1, K)).reshape(-1)
    edges = tuple(
        jnp.stack([i.reshape(-1), rows], axis=0) for i in (idx0, idx1)
    )
    return (out, (lp0, lp1), edges)
